# baseline (speedup 1.0000x reference)
import functools

import jax
import jax.numpy as jnp
import numpy as np
from jax import lax
from jax.experimental import pallas as pl
from jax.experimental.pallas import tpu as pltpu
from jax.experimental.pallas import tpu_sc as plsc

NC, NS = 2, 16
NW = NC * NS
B, S, D = 4096, 200, 300
DM = 256
DT = 64
PW = DM + DT
GM = DM // 32
GT = DT // 32
H = S // 2
RPW = B // NW
HPW = 2 * RPW


def _pool_body(x_hbm, main_hbm, tail_hbm, out_hbm,
               idx_v, bufm0, bufm1, buft0, buft1, acc_v, sem0, sem1):
    wid = lax.axis_index("s") * NC + lax.axis_index("c")
    base_h = wid * HPW
    pltpu.sync_copy(x_hbm.at[pl.ds(base_h, HPW)], idx_v)
    pltpu.async_copy(main_hbm.at[idx_v.at[0]], bufm0, sem0)
    pltpu.async_copy(tail_hbm.at[idx_v.at[0]], buft0, sem0)
    pltpu.async_copy(main_hbm.at[idx_v.at[1]], bufm1, sem1)
    pltpu.async_copy(tail_hbm.at[idx_v.at[1]], buft1, sem1)

    himask = jnp.full((16,), -65536, jnp.int32)

    def widen(v):
        lo = plsc.bitcast(lax.shift_left(v, 16), jnp.float32)
        hi = plsc.bitcast(lax.bitwise_and(v, himask), jnp.float32)
        return lo, hi

    def accum(bufm, buft, accs):
        def body(s, accs):
            out = list(accs)
            for g in range(GM):
                a, b = widen(bufm[s, pl.ds(g * 16, 16)])
                out[2 * g] = out[2 * g] + a
                out[2 * g + 1] = out[2 * g + 1] + b
            for t in range(GT):
                a, b = widen(buft[s, pl.ds(t * 16, 16)])
                out[2 * GM + 2 * t] = out[2 * GM + 2 * t] + a
                out[2 * GM + 2 * t + 1] = out[2 * GM + 2 * t + 1] + b
            return tuple(out)
        return lax.fori_loop(0, H, body, accs)

    def drain(g, bufm, buft, sem):
        pltpu.make_async_copy(main_hbm.at[idx_v.at[g]], bufm, sem).wait()
        pltpu.make_async_copy(tail_hbm.at[idx_v.at[g]], buft, sem).wait()

    def row_body(r, carry):
        g = 2 * r
        accs = tuple(jnp.zeros((16,), jnp.float32) for _ in range(PW // 16))

        drain(g, bufm0, buft0, sem0)
        accs = accum(bufm0, buft0, accs)

        @pl.when(r < RPW - 1)
        def _():
            pltpu.async_copy(main_hbm.at[idx_v.at[g + 2]], bufm0, sem0)
            pltpu.async_copy(tail_hbm.at[idx_v.at[g + 2]], buft0, sem0)

        drain(g + 1, bufm1, buft1, sem1)
        accs = accum(bufm1, buft1, accs)

        @pl.when(r < RPW - 1)
        def _():
            pltpu.async_copy(main_hbm.at[idx_v.at[g + 3]], bufm1, sem1)
            pltpu.async_copy(tail_hbm.at[idx_v.at[g + 3]], buft1, sem1)

        for c in range(PW // 16):
            acc_v[pl.ds(c * 16, 16)] = accs[c]
        pltpu.sync_copy(acc_v, out_hbm.at[wid * RPW + r])
        return carry

    lax.fori_loop(0, RPW, row_body, 0)


def _pooled_sums(x2, main, tail):
    return pl.kernel(
        _pool_body,
        out_type=jax.ShapeDtypeStruct((B, PW), jnp.float32),
        mesh=plsc.VectorSubcoreMesh(core_axis_name="c", subcore_axis_name="s"),
        scratch_types=[
            pltpu.VMEM((HPW, H), jnp.int32),
            pltpu.VMEM((H, DM // 2), jnp.int32),
            pltpu.VMEM((H, DM // 2), jnp.int32),
            pltpu.VMEM((H, DT // 2), jnp.int32),
            pltpu.VMEM((H, DT // 2), jnp.int32),
            pltpu.VMEM((PW,), jnp.float32),
            pltpu.SemaphoreType.DMA,
            pltpu.SemaphoreType.DMA,
        ],
        compiler_params=pltpu.CompilerParams(use_tc_tiling_on_sc=False,
                                             needs_layout_passes=False),
    )(x2, main, tail)


def _pooled_col_map():
    src = np.zeros(PW, np.int32)
    valid = np.zeros(PW, bool)
    for j in range(PW):
        g, k = divmod(j, 32)
        local = 2 * k if k < 16 else 2 * (k - 16) + 1
        col = g * 32 + local if g < GM else DM + (g - GM) * 32 + local
        src[j] = min(col, D - 1)
        valid[j] = col < D
    return src, valid

_SRC_COLS, _SRC_VALID = _pooled_col_map()


def _mlp_body(x_ref, w1_ref, b1_ref, w2_ref, b2_ref, w3_ref, b3_ref, o_ref):
    p = jax.lax.Precision.HIGHEST
    h = jnp.dot(x_ref[...], w1_ref[...], precision=p,
                preferred_element_type=jnp.float32)
    h = jnp.maximum(h + b1_ref[...], 0.0)
    h = jnp.dot(h, w2_ref[...], precision=p,
                preferred_element_type=jnp.float32)
    h = jnp.maximum(h + b2_ref[...], 0.0)
    o_ref[...] = jnp.dot(h, w3_ref[...], precision=p,
                         preferred_element_type=jnp.float32) + b3_ref[...]


def _mlp(pooled, W1p, b1, W2, b2, W3, b3):
    nb = 8
    bm = B // nb
    return pl.pallas_call(
        _mlp_body,
        grid=(nb,),
        in_specs=[
            pl.BlockSpec((bm, PW), lambda i: (i, 0)),
            pl.BlockSpec((PW, 256), lambda i: (0, 0)),
            pl.BlockSpec((1, 256), lambda i: (0, 0)),
            pl.BlockSpec((256, 128), lambda i: (0, 0)),
            pl.BlockSpec((1, 128), lambda i: (0, 0)),
            pl.BlockSpec((128, 32), lambda i: (0, 0)),
            pl.BlockSpec((1, 32), lambda i: (0, 0)),
        ],
        out_specs=pl.BlockSpec((bm, 32), lambda i: (i, 0)),
        out_shape=jax.ShapeDtypeStruct((B, 32), jnp.float32),
    )(pooled, W1p, b1.reshape(1, 256), W2, b2.reshape(1, 128),
      W3, b3.reshape(1, 32))


def kernel(x, table, W1, b1, W2, b2, W3, b3):
    x2 = x.astype(jnp.int32).reshape(B * 2, H)
    v = table.shape[0]
    main = lax.bitcast_convert_type(
        table[:, :DM].astype(jnp.bfloat16).reshape(v, DM // 2, 2), jnp.int32)
    tail = lax.bitcast_convert_type(
        jnp.pad(table[:, DM:].astype(jnp.bfloat16),
                ((0, 0), (0, PW - D))).reshape(v, DT // 2, 2), jnp.int32)
    W1s = W1 * (1.0 / S)
    W1p = jnp.where(jnp.asarray(_SRC_VALID)[:, None],
                    W1s[jnp.asarray(_SRC_COLS)], 0.0)
    pooled = _pooled_sums(x2, main, tail)
    return _mlp(pooled, W1p, b1, W2, b2, W3, b3)


# speedup vs baseline: 1.7307x; 1.7307x over previous
import functools

import jax
import jax.numpy as jnp
import numpy as np
from jax import lax
from jax.experimental import pallas as pl
from jax.experimental.pallas import tpu as pltpu
from jax.experimental.pallas import tpu_sc as plsc

NC, NS = 2, 16
NW = NC * NS
B, S, D = 4096, 200, 300
DM = 256
DT = 64
PW = DM + DT
GM = DM // 32
GT = DT // 32
H = S // 2
RPW = B // NW
HPW = 2 * RPW


def _pool_body(x_hbm, main_hbm, tail_hbm, out_hbm,
               idx_v, bufm0, bufm1, buft0, buft1, acc_v, sem0, sem1):
    wid = lax.axis_index("s") * NC + lax.axis_index("c")
    base_h = wid * HPW
    pltpu.sync_copy(x_hbm.at[pl.ds(base_h, HPW)], idx_v)
    pltpu.async_copy(main_hbm.at[idx_v.at[0]], bufm0, sem0)
    pltpu.async_copy(tail_hbm.at[idx_v.at[0]], buft0, sem0)
    pltpu.async_copy(main_hbm.at[idx_v.at[1]], bufm1, sem1)
    pltpu.async_copy(tail_hbm.at[idx_v.at[1]], buft1, sem1)

    himask = jnp.full((16,), -65536, jnp.int32)

    def widen(v):
        lo = plsc.bitcast(lax.shift_left(v, 16), jnp.float32)
        hi = plsc.bitcast(lax.bitwise_and(v, himask), jnp.float32)
        return lo, hi

    def accum(bufm, buft, accs):
        def body(s, accs):
            out = list(accs)
            for g in range(GM):
                v = plsc.bitcast(bufm[s, pl.ds(g * 32, 32)], jnp.int32)
                a, b = widen(v)
                out[2 * g] = out[2 * g] + a
                out[2 * g + 1] = out[2 * g + 1] + b
            for t in range(GT):
                v = plsc.bitcast(buft[s, pl.ds(t * 32, 32)], jnp.int32)
                a, b = widen(v)
                out[2 * GM + 2 * t] = out[2 * GM + 2 * t] + a
                out[2 * GM + 2 * t + 1] = out[2 * GM + 2 * t + 1] + b
            return tuple(out)
        return lax.fori_loop(0, H, body, accs)

    def drain(g, bufm, buft, sem):
        pltpu.make_async_copy(main_hbm.at[idx_v.at[g]], bufm, sem).wait()
        pltpu.make_async_copy(tail_hbm.at[idx_v.at[g]], buft, sem).wait()

    def row_body(r, carry):
        g = 2 * r
        accs = tuple(jnp.zeros((16,), jnp.float32) for _ in range(PW // 16))

        drain(g, bufm0, buft0, sem0)
        accs = accum(bufm0, buft0, accs)

        @pl.when(r < RPW - 1)
        def _():
            pltpu.async_copy(main_hbm.at[idx_v.at[g + 2]], bufm0, sem0)
            pltpu.async_copy(tail_hbm.at[idx_v.at[g + 2]], buft0, sem0)

        drain(g + 1, bufm1, buft1, sem1)
        accs = accum(bufm1, buft1, accs)

        @pl.when(r < RPW - 1)
        def _():
            pltpu.async_copy(main_hbm.at[idx_v.at[g + 3]], bufm1, sem1)
            pltpu.async_copy(tail_hbm.at[idx_v.at[g + 3]], buft1, sem1)

        for c in range(PW // 16):
            acc_v[pl.ds(c * 16, 16)] = accs[c]
        pltpu.sync_copy(acc_v, out_hbm.at[wid * RPW + r])
        return carry

    lax.fori_loop(0, RPW, row_body, 0)


def _pooled_sums(x2, main, tail):
    return pl.kernel(
        _pool_body,
        out_type=jax.ShapeDtypeStruct((B, PW), jnp.float32),
        mesh=plsc.VectorSubcoreMesh(core_axis_name="c", subcore_axis_name="s"),
        scratch_types=[
            pltpu.VMEM((HPW, H), jnp.int32),
            pltpu.VMEM((H, DM), jnp.bfloat16),
            pltpu.VMEM((H, DM), jnp.bfloat16),
            pltpu.VMEM((H, DT), jnp.bfloat16),
            pltpu.VMEM((H, DT), jnp.bfloat16),
            pltpu.VMEM((PW,), jnp.float32),
            pltpu.SemaphoreType.DMA,
            pltpu.SemaphoreType.DMA,
        ],
        compiler_params=pltpu.CompilerParams(use_tc_tiling_on_sc=False,
                                             needs_layout_passes=False),
    )(x2, main, tail)


def _pooled_col_map():
    src = np.zeros(PW, np.int32)
    valid = np.zeros(PW, bool)
    for j in range(PW):
        g, k = divmod(j, 32)
        local = 2 * k if k < 16 else 2 * (k - 16) + 1
        col = g * 32 + local if g < GM else DM + (g - GM) * 32 + local
        src[j] = min(col, D - 1)
        valid[j] = col < D
    return src, valid

_SRC_COLS, _SRC_VALID = _pooled_col_map()


def _mlp_body(x_ref, w1_ref, b1_ref, w2_ref, b2_ref, w3_ref, b3_ref, o_ref):
    p = jax.lax.Precision.HIGHEST
    h = jnp.dot(x_ref[...], w1_ref[...], precision=p,
                preferred_element_type=jnp.float32)
    h = jnp.maximum(h + b1_ref[...], 0.0)
    h = jnp.dot(h, w2_ref[...], precision=p,
                preferred_element_type=jnp.float32)
    h = jnp.maximum(h + b2_ref[...], 0.0)
    o_ref[...] = jnp.dot(h, w3_ref[...], precision=p,
                         preferred_element_type=jnp.float32) + b3_ref[...]


def _mlp(pooled, W1p, b1, W2, b2, W3, b3):
    nb = 8
    bm = B // nb
    return pl.pallas_call(
        _mlp_body,
        grid=(nb,),
        in_specs=[
            pl.BlockSpec((bm, PW), lambda i: (i, 0)),
            pl.BlockSpec((PW, 256), lambda i: (0, 0)),
            pl.BlockSpec((1, 256), lambda i: (0, 0)),
            pl.BlockSpec((256, 128), lambda i: (0, 0)),
            pl.BlockSpec((1, 128), lambda i: (0, 0)),
            pl.BlockSpec((128, 32), lambda i: (0, 0)),
            pl.BlockSpec((1, 32), lambda i: (0, 0)),
        ],
        out_specs=pl.BlockSpec((bm, 32), lambda i: (i, 0)),
        out_shape=jax.ShapeDtypeStruct((B, 32), jnp.float32),
    )(pooled, W1p, b1.reshape(1, 256), W2, b2.reshape(1, 128),
      W3, b3.reshape(1, 32))


def kernel(x, table, W1, b1, W2, b2, W3, b3):
    x2 = x.astype(jnp.int32).reshape(B * 2, H)
    t16 = table.astype(jnp.bfloat16)
    main = t16[:, :DM]
    tail = jnp.pad(t16[:, DM:], ((0, 0), (0, PW - D)))
    W1s = W1 * (1.0 / S)
    W1p = jnp.where(jnp.asarray(_SRC_VALID)[:, None],
                    W1s[jnp.asarray(_SRC_COLS)], 0.0)
    pooled = _pooled_sums(x2, main, tail)
    return _mlp(pooled, W1p, b1, W2, b2, W3, b3)
